# Initial kernel scaffold; baseline (speedup 1.0000x reference)
import jax
import jax.numpy as jnp
from jax.experimental import pallas as pl


def kernel(hidden_states, w_gate, w1, w2, ws_gate, ws_up, ws_down):
    raise NotImplementedError("write your pallas kernel here")


# speedup vs baseline: 2.1549x; 2.1549x over previous
import functools

import jax
import jax.numpy as jnp
from jax.experimental import pallas as pl
from jax.experimental.pallas import tpu as pltpu

E = 8
TOPK = 2
D = 1024
I = 2048
IS = 4096
S = 2048
N = S * TOPK
BT = 256
NB = N // BT + E
NPAD = NB * BT

_F32 = jnp.float32
_BF16 = jnp.bfloat16


def _router_body(x_ref, wgt_ref, i0_ref, i1_ref, s0_ref, s1_ref):
    logits = jnp.dot(x_ref[...], wgt_ref[...], preferred_element_type=_F32)
    iota = jax.lax.broadcasted_iota(jnp.int32, (S, E), 1)
    v0 = jnp.max(logits, axis=1, keepdims=True)
    i0 = jnp.min(jnp.where(logits == v0, iota, E), axis=1, keepdims=True)
    masked = jnp.where(iota == i0, -jnp.inf, logits)
    v1 = jnp.max(masked, axis=1, keepdims=True)
    i1 = jnp.min(jnp.where(masked == v1, iota, E), axis=1, keepdims=True)
    s0 = 1.0 / (1.0 + jnp.exp(v1 - v0))
    i0_ref[...] = i0
    i1_ref[...] = i1
    s0_ref[...] = s0
    s1_ref[...] = 1.0 - s0


def _router(xf, w_gate_t):
    return pl.pallas_call(
        _router_body,
        out_shape=(
            jax.ShapeDtypeStruct((S, 1), jnp.int32),
            jax.ShapeDtypeStruct((S, 1), jnp.int32),
            jax.ShapeDtypeStruct((S, 1), _F32),
            jax.ShapeDtypeStruct((S, 1), _F32),
        ),
    )(xf, w_gate_t)


def _expert_body(be_ref, ids_ref, xf_ref, w1_ref, w2_ref, out_ref):
    ids = ids_ref[0]
    iota = jax.lax.broadcasted_iota(jnp.int32, (BT, S), 1)
    onehot = (ids == iota).astype(_BF16)
    xg = jnp.dot(onehot, xf_ref[...], preferred_element_type=_F32)
    h = jnp.dot(xg.astype(_BF16), w1_ref[0], preferred_element_type=_F32)
    a = h[:, :I]
    b = h[:, I:]
    act = (a * jax.nn.sigmoid(a) * b).astype(_BF16)
    out_ref[...] = jnp.dot(act, w2_ref[0],
                           preferred_element_type=_F32).astype(_BF16)


def _expert_mlp(block_expert, row_ids, xf_bf, w1_bf, w2_bf):
    grid_spec = pltpu.PrefetchScalarGridSpec(
        num_scalar_prefetch=1,
        grid=(NB,),
        in_specs=[
            pl.BlockSpec((1, BT, 1), lambda g, be: (g, 0, 0)),
            pl.BlockSpec((S, D), lambda g, be: (0, 0)),
            pl.BlockSpec((1, D, 2 * I), lambda g, be: (be[g], 0, 0)),
            pl.BlockSpec((1, I, D), lambda g, be: (be[g], 0, 0)),
        ],
        out_specs=pl.BlockSpec((BT, D), lambda g, be: (g, 0)),
    )
    return pl.pallas_call(
        _expert_body,
        grid_spec=grid_spec,
        out_shape=jax.ShapeDtypeStruct((NPAD, D), _BF16),
        compiler_params=pltpu.CompilerParams(
            dimension_semantics=("arbitrary",),
        ),
    )(block_expert, row_ids, xf_bf, w1_bf, w2_bf)


def _combine_body(x_ref, eo_ref, p0_ref, p1_ref, s0_ref, s1_ref,
                  wg_ref, wu_ref, wd_ref, o_ref):
    x = x_ref[...]
    g_ = jnp.dot(x, wg_ref[...], preferred_element_type=_F32)
    u_ = jnp.dot(x, wu_ref[...], preferred_element_type=_F32)
    act = (g_ * jax.nn.sigmoid(g_) * u_).astype(_BF16)
    shared = jnp.dot(act, wd_ref[...], preferred_element_type=_F32)

    p0 = p0_ref[0]
    p1 = p1_ref[0]
    s0 = s0_ref[0]
    s1 = s1_ref[0]
    iota = jax.lax.broadcasted_iota(jnp.int32, (BT, NPAD), 1)
    comb = (jnp.where(iota == p0, s0, 0.0)
            + jnp.where(iota == p1, s1, 0.0)).astype(_BF16)
    moe = jnp.dot(comb, eo_ref[...], preferred_element_type=_F32)
    o_ref[...] = shared + moe


def _combine_shared(xf_bf, eo, p0, p1, s0, s1, wsg_bf, wsu_bf, wsd_bf):
    tb = S // BT
    return pl.pallas_call(
        _combine_body,
        grid=(tb,),
        in_specs=[
            pl.BlockSpec((BT, D), lambda t: (t, 0)),
            pl.BlockSpec((NPAD, D), lambda t: (0, 0)),
            pl.BlockSpec((1, BT, 1), lambda t: (t, 0, 0)),
            pl.BlockSpec((1, BT, 1), lambda t: (t, 0, 0)),
            pl.BlockSpec((1, BT, 1), lambda t: (t, 0, 0)),
            pl.BlockSpec((1, BT, 1), lambda t: (t, 0, 0)),
            pl.BlockSpec((D, IS), lambda t: (0, 0)),
            pl.BlockSpec((D, IS), lambda t: (0, 0)),
            pl.BlockSpec((IS, D), lambda t: (0, 0)),
        ],
        out_specs=pl.BlockSpec((BT, D), lambda t: (t, 0)),
        out_shape=jax.ShapeDtypeStruct((S, D), _F32),
        compiler_params=pltpu.CompilerParams(
            dimension_semantics=("arbitrary",),
        ),
    )(xf_bf, eo, p0, p1, s0, s1, wsg_bf, wsu_bf, wsd_bf)


@functools.partial(jax.jit, static_argnums=())
def kernel(hidden_states, w_gate, w1, w2, ws_gate, ws_up, ws_down):
    shape = hidden_states.shape
    xf = hidden_states.reshape(-1, D)

    i0, i1, s0, s1 = _router(xf, w_gate.T)

    flat = jnp.stack([i0[:, 0], i1[:, 0]], axis=1).reshape(-1)
    onehot = (flat[:, None] == jnp.arange(E)[None, :]).astype(jnp.int32)
    cum = jnp.cumsum(onehot, axis=0)
    counts = cum[-1]
    rank = jnp.take_along_axis(cum, flat[:, None], axis=1)[:, 0] - 1
    blocks_e = (counts + BT - 1) // BT
    ends_blocks = jnp.cumsum(blocks_e)
    base_rows = (ends_blocks - blocks_e) * BT
    dest = base_rows[flat] + rank
    block_expert = jnp.clip(
        jnp.searchsorted(ends_blocks, jnp.arange(NB), side="right"),
        0, E - 1).astype(jnp.int32)
    row_ids = jnp.zeros((NPAD,), jnp.int32).at[dest].set(
        jnp.arange(N, dtype=jnp.int32) // TOPK)
    p0 = dest[0::2].reshape(S // BT, BT, 1).astype(jnp.int32)
    p1 = dest[1::2].reshape(S // BT, BT, 1).astype(jnp.int32)

    xf_bf = xf.astype(_BF16)
    eo = _expert_mlp(block_expert, row_ids.reshape(NB, BT, 1),
                     xf_bf, w1.astype(_BF16), w2.astype(_BF16))

    s0b = s0.reshape(S // BT, BT, 1)
    s1b = s1.reshape(S // BT, BT, 1)
    out = _combine_shared(xf_bf, eo, p0, p1, s0b, s1b,
                          ws_gate.astype(_BF16), ws_up.astype(_BF16),
                          ws_down.astype(_BF16))
    return out.reshape(shape)
